# baseline (speedup 1.0000x reference)
import functools

import jax
import jax.numpy as jnp
from jax import lax
from jax.experimental import pallas as pl
from jax.experimental.pallas import tpu as pltpu
from jax.experimental.pallas import tpu_sc as plsc

_N = 20000
_B = 512
_SENTINEL = 1e9


def _make_sc_gather(n, npad):
    info = plsc.get_sparse_core_info()
    nworkers = info.num_cores * info.num_subcores
    cpw = npad // nworkers
    assert cpw % 16 == 0 and (cpw * 4) % 8 == 0
    groups = cpw // 16
    mesh = plsc.VectorSubcoreMesh(core_axis_name="c", subcore_axis_name="s")

    @functools.partial(
        pl.kernel,
        out_type=jax.ShapeDtypeStruct((4, npad), jnp.float32),
        mesh=mesh,
        compiler_params=pltpu.CompilerParams(needs_layout_passes=False),
        scratch_types=[
            pltpu.VMEM((4 * n + 16,), jnp.float32),
            pltpu.VMEM((cpw,), jnp.int32),
            pltpu.VMEM((cpw,), jnp.float32),
            pltpu.VMEM((cpw,), jnp.float32),
            pltpu.VMEM((cpw,), jnp.float32),
            pltpu.VMEM((cpw,), jnp.float32),
        ],
    )
    def sc_gather(flat_hbm, order_hbm, out_hbm, table_v, idx_v, o0, o1, o2, o3):
        wid = lax.axis_index("s") * info.num_cores + lax.axis_index("c")
        base = wid * cpw
        pltpu.sync_copy(flat_hbm, table_v)
        pltpu.sync_copy(order_hbm.at[pl.ds(base, cpw)], idx_v)
        outs = (o0, o1, o2, o3)
        for g in range(groups):
            idx16 = idx_v[pl.ds(g * 16, 16)]
            fl = idx16 * 4
            for c in range(4):
                outs[c][pl.ds(g * 16, 16)] = plsc.load_gather(
                    table_v, [fl + c]
                )
        for c in range(4):
            pltpu.sync_copy(outs[c], out_hbm.at[c, pl.ds(base, cpw)])

    return sc_gather


def _nms_kernel(coordsT_ref, outT_ref, presup_ref, *, nblk, blk):
    B = blk
    presup_ref[...] = jnp.zeros_like(presup_ref)

    def row_block(k, _):
        s = k * B
        cx1 = coordsT_ref[0:1, pl.ds(s, B)]
        cy1 = coordsT_ref[1:2, pl.ds(s, B)]
        cx2 = coordsT_ref[2:3, pl.ds(s, B)]
        cy2 = coordsT_ref[3:4, pl.ds(s, B)]
        carea = (cx2 - cx1) * (cy2 - cy1)
        px1 = jnp.transpose(cx1)
        py1 = jnp.transpose(cy1)
        px2 = jnp.transpose(cx2)
        py2 = jnp.transpose(cy2)

        x1 = jnp.maximum(cx1, px1)
        y1 = jnp.maximum(cy1, py1)
        x2 = jnp.minimum(cx2, px2)
        y2 = jnp.minimum(cy2, py2)
        inter = jnp.maximum(x2 - x1, 0.0) * jnp.maximum(y2 - y1, 0.0)
        denom = (carea + jnp.transpose(carea)) - inter + 1e-9
        m_diag = (inter + inter) > denom
        rows = lax.broadcasted_iota(jnp.int32, (B, B), 0)
        cols = lax.broadcasted_iota(jnp.int32, (B, B), 1)
        mu = jnp.where(m_diag & (rows < cols), 1.0, 0.0)

        def lead_cond(state):
            cand, _ = state
            return jnp.any(cand > 0.0)

        def lead_body(state):
            cand, kept = state
            supp_cnt = jnp.dot(cand, mu, preferred_element_type=jnp.float32)
            leader = jnp.where(supp_cnt > 0.0, 0.0, cand)
            rem_cnt = jnp.dot(leader, mu, preferred_element_type=jnp.float32)
            kept = kept + leader
            cand = jnp.where((rem_cnt > 0.0) | (leader > 0.0), 0.0, cand)
            return cand, kept

        pre = presup_ref[0:1, pl.ds(s, B)]
        cand0 = jnp.where(pre > 0.0, 0.0, 1.0)
        _, kept = lax.while_loop(
            lead_cond, lead_body, (cand0, jnp.zeros((1, B), jnp.float32))
        )

        outT_ref[:, pl.ds(s, B)] = coordsT_ref[:, pl.ds(s, B)] * kept

        keptc = jnp.transpose(kept) > 0.0
        zbb = jnp.zeros((B, B), jnp.float32)
        sx1 = jnp.where(keptc, px1, _SENTINEL) + zbb
        sy1 = jnp.where(keptc, py1, _SENTINEL) + zbb
        sx2 = jnp.where(keptc, px2, _SENTINEL) + zbb
        sy2 = jnp.where(keptc, py2, _SENTINEL) + zbb
        sarea = (sx2 - sx1) * (sy2 - sy1)

        def see_block(m, _):
            t = m * B
            ex1 = coordsT_ref[0:1, pl.ds(t, B)]
            ey1 = coordsT_ref[1:2, pl.ds(t, B)]
            ex2 = coordsT_ref[2:3, pl.ds(t, B)]
            ey2 = coordsT_ref[3:4, pl.ds(t, B)]
            earea = (ex2 - ex1) * (ey2 - ey1)
            acc = jnp.zeros((8, B), jnp.float32)
            for c in range(B // 8):
                r = c * 8
                ux1 = lax.slice(sx1, (r, 0), (r + 8, B))
                uy1 = lax.slice(sy1, (r, 0), (r + 8, B))
                ux2 = lax.slice(sx2, (r, 0), (r + 8, B))
                uy2 = lax.slice(sy2, (r, 0), (r + 8, B))
                uarea = lax.slice(sarea, (r, 0), (r + 8, B))
                a1 = jnp.maximum(ex1, ux1)
                b1 = jnp.maximum(ey1, uy1)
                a2 = jnp.minimum(ex2, ux2)
                b2 = jnp.minimum(ey2, uy2)
                intr = jnp.maximum(a2 - a1, 0.0) * jnp.maximum(b2 - b1, 0.0)
                dnm = (earea + uarea) - intr + 1e-9
                acc = jnp.maximum(
                    acc, jnp.where((intr + intr) > dnm, 1.0, 0.0)
                )
            sup = jnp.max(acc, axis=0, keepdims=True)
            old = presup_ref[0:1, pl.ds(t, B)]
            presup_ref[0:1, pl.ds(t, B)] = jnp.maximum(old, sup)
            return 0

        lax.fori_loop(k + 1, nblk, see_block, 0)
        return 0

    lax.fori_loop(0, nblk, row_block, 0)


def _nms_sorted(coordsT, nblk, blk, interpret=False):
    npad = nblk * blk
    return pl.pallas_call(
        functools.partial(_nms_kernel, nblk=nblk, blk=blk),
        out_shape=jax.ShapeDtypeStruct((4, npad), jnp.float32),
        scratch_shapes=[pltpu.VMEM((1, npad), jnp.float32)],
        interpret=interpret,
    )(coordsT)


def kernel(boxes, scores, interpret=False):
    n = boxes.shape[0]
    blk = _B
    nblk = (n + blk - 1) // blk
    npad = nblk * blk
    order = jnp.argsort(-scores)
    if interpret:
        boxes_sorted = boxes[order]
        coordsT = jnp.pad(boxes_sorted, ((0, npad - n), (0, 0))).T
    else:
        order_padded = jnp.pad(
            order.astype(jnp.int32), (0, npad - n), constant_values=n
        )
        flat = jnp.concatenate(
            [boxes.reshape(-1), jnp.zeros((16,), jnp.float32)]
        )
        coordsT = _make_sc_gather(n, npad)(flat, order_padded)
    outT = _nms_sorted(coordsT, nblk, blk, interpret=interpret)
    return outT.T[:n]


# speedup vs baseline: 1.0342x; 1.0342x over previous
import functools

import jax
import jax.numpy as jnp
from jax import lax
from jax.experimental import pallas as pl
from jax.experimental.pallas import tpu as pltpu
from jax.experimental.pallas import tpu_sc as plsc

_N = 20000
_B = 512
_SENTINEL = 1e9


def _make_sc_gather(n, npad):
    info = plsc.get_sparse_core_info()
    nworkers = info.num_cores * info.num_subcores
    cpw = npad // nworkers
    assert cpw % 16 == 0 and (cpw * 4) % 8 == 0
    groups = cpw // 16
    mesh = plsc.VectorSubcoreMesh(core_axis_name="c", subcore_axis_name="s")

    @functools.partial(
        pl.kernel,
        out_type=jax.ShapeDtypeStruct((4, npad), jnp.float32),
        mesh=mesh,
        compiler_params=pltpu.CompilerParams(needs_layout_passes=False),
        scratch_types=[
            pltpu.VMEM((4 * n + 16,), jnp.float32),
            pltpu.VMEM((cpw,), jnp.int32),
            pltpu.VMEM((cpw,), jnp.float32),
            pltpu.VMEM((cpw,), jnp.float32),
            pltpu.VMEM((cpw,), jnp.float32),
            pltpu.VMEM((cpw,), jnp.float32),
        ],
    )
    def sc_gather(flat_hbm, order_hbm, out_hbm, table_v, idx_v, o0, o1, o2, o3):
        wid = lax.axis_index("s") * info.num_cores + lax.axis_index("c")
        base = wid * cpw
        pltpu.sync_copy(flat_hbm, table_v)
        pltpu.sync_copy(order_hbm.at[pl.ds(base, cpw)], idx_v)
        outs = (o0, o1, o2, o3)
        for g in range(groups):
            idx16 = idx_v[pl.ds(g * 16, 16)]
            fl = idx16 * 4
            for c in range(4):
                outs[c][pl.ds(g * 16, 16)] = plsc.load_gather(
                    table_v, [fl + c]
                )
        for c in range(4):
            pltpu.sync_copy(outs[c], out_hbm.at[c, pl.ds(base, cpw)])

    return sc_gather


def _nms_kernel(coordsT_ref, outT_ref, presup_ref, *, nblk, blk):
    B = blk
    presup_ref[...] = jnp.zeros_like(presup_ref)

    def row_block(k, _):
        s = k * B
        cx1 = coordsT_ref[0:1, pl.ds(s, B)]
        cy1 = coordsT_ref[1:2, pl.ds(s, B)]
        cx2 = coordsT_ref[2:3, pl.ds(s, B)]
        cy2 = coordsT_ref[3:4, pl.ds(s, B)]
        carea = (cx2 - cx1) * (cy2 - cy1)
        px1 = jnp.transpose(cx1)
        py1 = jnp.transpose(cy1)
        px2 = jnp.transpose(cx2)
        py2 = jnp.transpose(cy2)

        x1 = jnp.maximum(cx1, px1)
        y1 = jnp.maximum(cy1, py1)
        x2 = jnp.minimum(cx2, px2)
        y2 = jnp.minimum(cy2, py2)
        inter = jnp.maximum(x2 - x1, 0.0) * jnp.maximum(y2 - y1, 0.0)
        denom = (carea + jnp.transpose(carea)) - inter + 1e-9
        m_diag = (inter + inter) > denom
        rows = lax.broadcasted_iota(jnp.int32, (B, B), 0)
        cols = lax.broadcasted_iota(jnp.int32, (B, B), 1)
        mu = jnp.where(m_diag & (rows < cols), 1.0, 0.0)

        def lead_cond(state):
            cand, _ = state
            return jnp.any(cand > 0.0)

        def lead_body(state):
            cand, kept = state
            supp_cnt = jnp.dot(cand, mu, preferred_element_type=jnp.float32)
            leader = jnp.where(supp_cnt > 0.0, 0.0, cand)
            rem_cnt = jnp.dot(leader, mu, preferred_element_type=jnp.float32)
            kept = kept + leader
            cand = jnp.where((rem_cnt > 0.0) | (leader > 0.0), 0.0, cand)
            return cand, kept

        pre = presup_ref[0:1, pl.ds(s, B)]
        cand0 = jnp.where(pre > 0.0, 0.0, 1.0)
        _, kept = lax.while_loop(
            lead_cond, lead_body, (cand0, jnp.zeros((1, B), jnp.float32))
        )

        outT_ref[:, pl.ds(s, B)] = coordsT_ref[:, pl.ds(s, B)] * kept

        keptc = jnp.transpose(kept) > 0.0
        zbb = jnp.zeros((B, B), jnp.float32)
        sx1 = jnp.where(keptc, px1, _SENTINEL) + zbb
        sy1 = jnp.where(keptc, py1, _SENTINEL) + zbb
        sx2 = jnp.where(keptc, px2, _SENTINEL) + zbb
        sy2 = jnp.where(keptc, py2, _SENTINEL) + zbb
        sarea = (sx2 - sx1) * (sy2 - sy1)

        def see_block(m, _):
            t = m * B
            ex1 = coordsT_ref[0:1, pl.ds(t, B)]
            ey1 = coordsT_ref[1:2, pl.ds(t, B)]
            ex2 = coordsT_ref[2:3, pl.ds(t, B)]
            ey2 = coordsT_ref[3:4, pl.ds(t, B)]
            earea = (ex2 - ex1) * (ey2 - ey1)
            acc = jnp.zeros((8, B), jnp.bool_)
            for c in range(B // 8):
                r = c * 8
                ux1 = lax.slice(sx1, (r, 0), (r + 8, B))
                uy1 = lax.slice(sy1, (r, 0), (r + 8, B))
                ux2 = lax.slice(sx2, (r, 0), (r + 8, B))
                uy2 = lax.slice(sy2, (r, 0), (r + 8, B))
                uarea = lax.slice(sarea, (r, 0), (r + 8, B))
                a1 = jnp.maximum(ex1, ux1)
                b1 = jnp.maximum(ey1, uy1)
                a2 = jnp.minimum(ex2, ux2)
                b2 = jnp.minimum(ey2, uy2)
                intr = jnp.maximum(a2 - a1, 0.0) * jnp.maximum(b2 - b1, 0.0)
                dnm = (earea + uarea) - intr + 1e-9
                acc = acc | ((intr + intr) > dnm)
            sup = jnp.any(acc, axis=0, keepdims=True)
            old = presup_ref[0:1, pl.ds(t, B)]
            presup_ref[0:1, pl.ds(t, B)] = jnp.maximum(
                old, sup.astype(jnp.float32)
            )
            return 0

        lax.fori_loop(k + 1, nblk, see_block, 0)
        return 0

    lax.fori_loop(0, nblk, row_block, 0)


def _nms_sorted(coordsT, nblk, blk, interpret=False):
    npad = nblk * blk
    return pl.pallas_call(
        functools.partial(_nms_kernel, nblk=nblk, blk=blk),
        out_shape=jax.ShapeDtypeStruct((4, npad), jnp.float32),
        scratch_shapes=[pltpu.VMEM((1, npad), jnp.float32)],
        interpret=interpret,
    )(coordsT)


def kernel(boxes, scores, interpret=False):
    n = boxes.shape[0]
    blk = _B
    nblk = (n + blk - 1) // blk
    npad = nblk * blk
    order = jnp.argsort(-scores)
    if interpret:
        boxes_sorted = boxes[order]
        coordsT = jnp.pad(boxes_sorted, ((0, npad - n), (0, 0))).T
    else:
        order_padded = jnp.pad(
            order.astype(jnp.int32), (0, npad - n), constant_values=n
        )
        flat = jnp.concatenate(
            [boxes.reshape(-1), jnp.zeros((16,), jnp.float32)]
        )
        coordsT = _make_sc_gather(n, npad)(flat, order_padded)
    outT = _nms_sorted(coordsT, nblk, blk, interpret=interpret)
    return outT.T[:n]
